# baseline (speedup 1.0000x reference)
import functools

import jax
import jax.numpy as jnp
from jax import lax
from jax.experimental import pallas as pl
from jax.experimental.pallas import tpu as pltpu
from jax.experimental.pallas import tpu_sc as plsc

NC = 2
NS = 16
NW = NC * NS
LANE = 128
CH = 2560
NR = CH // LANE
MAGIC = 12582912.0


def _make_tc_project(B, Npad, C, H, W):
    rows = (B * Npad) // LANE
    rows_per_batch = Npad // LANE
    zero_idx = B * H * W
    hW = float(0.5 * W)
    hH = float(0.5 * H)

    def body(x0_ref, x1_ref, z_ref, idx_ref, valid_ref):
        zf = z_ref[...]
        a = x0_ref[...].astype(jnp.bfloat16).astype(jnp.float32)
        b = x1_ref[...].astype(jnp.bfloat16).astype(jnp.float32)
        z = zf.astype(jnp.bfloat16).astype(jnp.float32)
        uf = (a * hW + z * hW) / z
        vf = (b * hH + z * hH) / z
        ur = (uf + MAGIC) - MAGIC
        vr = (vf + MAGIC) - MAGIC
        cond = ((ur > 0.0) & (ur < W) & (vr > 0.0) & (vr < H) & (zf > 0.0))
        ui = ur.astype(jnp.int32)
        vi = vr.astype(jnp.int32)
        rid = lax.broadcasted_iota(jnp.int32, (rows, LANE), 0)
        tbase = (rid // rows_per_batch) * (H * W)
        ind = ui + vi * H + tbase
        idx_ref[...] = jnp.where(cond, ind, zero_idx)
        valid_ref[...] = jnp.where(cond, 1, 0).astype(jnp.int32)

    return pl.pallas_call(
        body,
        out_shape=(
            jax.ShapeDtypeStruct((rows, LANE), jnp.int32),
            jax.ShapeDtypeStruct((rows, LANE), jnp.int32),
        ),
    )


def _make_sc_gather(B, Npad, CW, n_table_rows):
    PW = (B * Npad) // NW
    CHUNKS = PW // CH
    ROWS_W = PW // LANE
    mesh = plsc.VectorSubcoreMesh(core_axis_name="c", subcore_axis_name="s")

    @functools.partial(
        pl.kernel,
        mesh=mesh,
        out_type=jax.ShapeDtypeStruct((B * Npad, CW), jnp.float32),
        scratch_types=[
            pltpu.VMEM((CH,), jnp.int32),
            pltpu.VMEM((CH, CW), jnp.float32),
            pltpu.SemaphoreType.DMA,
        ],
        compiler_params=pltpu.CompilerParams(use_tc_tiling_on_sc=False),
    )
    def sc_kernel(tab_hbm, idx_hbm, feat_hbm, idxv, featv, sem):
        wid = lax.axis_index("s") * NC + lax.axis_index("c")
        base = wid * PW

        def chunk_body(ci, carry):
            off = base + ci * CH
            pltpu.sync_copy(idx_hbm.at[pl.ds(off, CH)], idxv)
            pltpu.async_copy(tab_hbm.at[idxv], featv, sem).wait()
            pltpu.sync_copy(featv, feat_hbm.at[pl.ds(off, CH)])
            return carry

        lax.fori_loop(0, CHUNKS, chunk_body, 0)

    return sc_kernel


def kernel(x, img):
    B, N, _ = x.shape
    _, C, H, W = img.shape

    WPB = NW // B
    per_worker = -(-N // WPB)
    per_worker = -(-per_worker // CH) * CH
    Npad = per_worker * WPB

    xp = jnp.pad(x, ((0, 0), (0, Npad - N), (0, 0)))
    rows = (B * Npad) // LANE
    x0 = xp[..., 0].reshape(rows, LANE)
    x1 = xp[..., 1].reshape(rows, LANE)
    x2 = xp[..., 2].reshape(rows, LANE)

    CW = C + 1
    imgT = jnp.swapaxes(img.reshape(B, C, H * W), 1, 2).reshape(B * H * W, C)
    table = jnp.pad(imgT, ((0, 8), (0, CW - C)))

    idx, valid2d = _make_tc_project(B, Npad, C, H, W)(x0, x1, x2)
    feat_pad = _make_sc_gather(B, Npad, CW, table.shape[0])(
        table, idx.reshape(-1))

    feat = feat_pad.reshape(B, Npad, CW)[:, :N, :C]
    valid = valid2d.reshape(B, Npad)[:, :N, None].astype(jnp.int64)
    return (feat, valid)


# speedup vs baseline: 1.3590x; 1.3590x over previous
import functools

import jax
import jax.numpy as jnp
from jax import lax
from jax.experimental import pallas as pl
from jax.experimental.pallas import tpu as pltpu
from jax.experimental.pallas import tpu_sc as plsc

NC = 2
NS = 16
NW = NC * NS
LANE = 128
CH = 2560
NR = CH // LANE
MAGIC = 12582912.0


def _make_tc_project(B, Npad, C, H, W):
    rows = (B * Npad) // LANE
    rows_per_batch = Npad // LANE
    zero_idx = B * H * W
    hW = float(0.5 * W)
    hH = float(0.5 * H)

    def body(x0_ref, x1_ref, z_ref, idx_ref, valid_ref):
        zf = z_ref[...]
        a = x0_ref[...].astype(jnp.bfloat16).astype(jnp.float32)
        b = x1_ref[...].astype(jnp.bfloat16).astype(jnp.float32)
        z = zf.astype(jnp.bfloat16).astype(jnp.float32)
        uf = (a * hW + z * hW) / z
        vf = (b * hH + z * hH) / z
        ur = (uf + MAGIC) - MAGIC
        vr = (vf + MAGIC) - MAGIC
        cond = ((ur > 0.0) & (ur < W) & (vr > 0.0) & (vr < H) & (zf > 0.0))
        ui = ur.astype(jnp.int32)
        vi = vr.astype(jnp.int32)
        rid = lax.broadcasted_iota(jnp.int32, (rows, LANE), 0)
        tbase = (rid // rows_per_batch) * (H * W)
        ind = ui + vi * H + tbase
        idx_ref[...] = jnp.where(cond, ind, zero_idx)
        valid_ref[...] = jnp.where(cond, 1, 0).astype(jnp.int32)

    return pl.pallas_call(
        body,
        out_shape=(
            jax.ShapeDtypeStruct((rows, LANE), jnp.int32),
            jax.ShapeDtypeStruct((rows, LANE), jnp.int32),
        ),
    )


def _make_sc_gather(B, Npad, CW):
    PW = (B * Npad) // NW
    CHUNKS = PW // CH
    mesh = plsc.VectorSubcoreMesh(core_axis_name="c", subcore_axis_name="s")

    @functools.partial(
        pl.kernel,
        mesh=mesh,
        out_type=jax.ShapeDtypeStruct((B * Npad, CW), jnp.float32),
        scratch_types=[
            pltpu.VMEM((NR, LANE), jnp.int32),
            pltpu.VMEM((CH, CW), jnp.float32),
            pltpu.SemaphoreType.DMA,
        ],
        compiler_params=pltpu.CompilerParams(use_tc_tiling_on_sc=False),
    )
    def sc_kernel(tab_hbm, idx_hbm, feat_hbm, idxv, featv, sem):
        wid = lax.axis_index("s") * NC + lax.axis_index("c")
        base = wid * PW
        rbase = base // LANE

        def chunk_body(ci, carry):
            off = base + ci * CH
            pltpu.sync_copy(idx_hbm.at[pl.ds(rbase + ci * NR, NR)], idxv)
            cps = [
                pltpu.async_copy(tab_hbm.at[idxv.at[r]],
                                 featv.at[pl.ds(r * LANE, LANE)], sem)
                for r in range(NR)
            ]
            for cp in cps:
                cp.wait()
            pltpu.sync_copy(featv, feat_hbm.at[pl.ds(off, CH)])
            return carry

        lax.fori_loop(0, CHUNKS, chunk_body, 0)

    return sc_kernel


def kernel(x, img):
    B, N, _ = x.shape
    _, C, H, W = img.shape

    WPB = NW // B
    per_worker = -(-N // WPB)
    per_worker = -(-per_worker // CH) * CH
    Npad = per_worker * WPB

    xp = jnp.pad(x, ((0, 0), (0, Npad - N), (0, 0)))
    rows = (B * Npad) // LANE
    x0 = xp[..., 0].reshape(rows, LANE)
    x1 = xp[..., 1].reshape(rows, LANE)
    x2 = xp[..., 2].reshape(rows, LANE)

    CW = C
    imgT = jnp.swapaxes(img.reshape(B, C, H * W), 1, 2).reshape(B * H * W, C)
    table = jnp.pad(imgT, ((0, 8), (0, 0)))

    idx, valid2d = _make_tc_project(B, Npad, C, H, W)(x0, x1, x2)
    feat_pad = _make_sc_gather(B, Npad, CW)(table, idx)

    feat = feat_pad.reshape(B, Npad, CW)[:, :N, :C]
    valid = valid2d.reshape(B, Npad)[:, :N, None].astype(jnp.int64)
    return (feat, valid)


# speedup vs baseline: 1.4270x; 1.0500x over previous
import functools

import jax
import jax.numpy as jnp
from jax import lax
from jax.experimental import pallas as pl
from jax.experimental.pallas import tpu as pltpu
from jax.experimental.pallas import tpu_sc as plsc

NC = 2
NS = 16
NW = NC * NS
LANE = 128
CH = 2560
NR = CH // LANE
MAGIC = 12582912.0


def _make_tc_project(B, Npad, C, H, W):
    rows = (B * Npad) // LANE
    rows_per_batch = Npad // LANE
    zero_idx = B * H * W
    hW = float(0.5 * W)
    hH = float(0.5 * H)

    def body(x0_ref, x1_ref, z_ref, idx_ref, valid_ref):
        zf = z_ref[...]
        a = x0_ref[...].astype(jnp.bfloat16).astype(jnp.float32)
        b = x1_ref[...].astype(jnp.bfloat16).astype(jnp.float32)
        z = zf.astype(jnp.bfloat16).astype(jnp.float32)
        uf = (a * hW + z * hW) / z
        vf = (b * hH + z * hH) / z
        ur = (uf + MAGIC) - MAGIC
        vr = (vf + MAGIC) - MAGIC
        cond = ((ur > 0.0) & (ur < W) & (vr > 0.0) & (vr < H) & (zf > 0.0))
        ui = ur.astype(jnp.int32)
        vi = vr.astype(jnp.int32)
        rid = lax.broadcasted_iota(jnp.int32, (rows, LANE), 0)
        tbase = (rid // rows_per_batch) * (H * W)
        ind = ui + vi * H + tbase
        idx_ref[...] = jnp.where(cond, ind, zero_idx)
        valid_ref[...] = jnp.where(cond, 1, 0).astype(jnp.int32)

    return pl.pallas_call(
        body,
        out_shape=(
            jax.ShapeDtypeStruct((rows, LANE), jnp.int32),
            jax.ShapeDtypeStruct((rows, LANE), jnp.int32),
        ),
    )


def _make_sc_gather(B, Npad, N, CW):
    PW = (B * Npad) // NW
    CHUNKS = PW // CH
    WPB = NW // B
    mesh = plsc.VectorSubcoreMesh(core_axis_name="c", subcore_axis_name="s")

    @functools.partial(
        pl.kernel,
        mesh=mesh,
        out_type=jax.ShapeDtypeStruct((B * N, CW), jnp.float32),
        scratch_types=[
            pltpu.VMEM((NR, LANE), jnp.int32),
            pltpu.VMEM((NR, LANE), jnp.int32),
            pltpu.VMEM((CH, CW), jnp.float32),
            pltpu.VMEM((CH, CW), jnp.float32),
            pltpu.SemaphoreType.DMA,
            pltpu.SemaphoreType.DMA,
            pltpu.SemaphoreType.DMA,
        ],
        compiler_params=pltpu.CompilerParams(use_tc_tiling_on_sc=False),
    )
    def sc_kernel(tab_hbm, idx_hbm, feat_hbm, idxv0, idxv1, featv0, featv1,
                  sem_i, sem_g, sem_o):
        wid = lax.axis_index("s") * NC + lax.axis_index("c")
        rbase = (wid * PW) // LANE
        b = wid // WPB
        q = wid % WPB
        obase = b * N + q * PW
        ivs = (idxv0, idxv1)
        fvs = (featv0, featv1)

        def load_idx(ci):
            return pltpu.async_copy(
                idx_hbm.at[pl.ds(rbase + ci * NR, NR)], ivs[ci % 2], sem_i)

        def fire_gather(ci):
            iv, fv = ivs[ci % 2], fvs[ci % 2]
            return [
                pltpu.async_copy(tab_hbm.at[iv.at[r]],
                                 fv.at[pl.ds(r * LANE, LANE)], sem_g)
                for r in range(NR)
            ]

        def writeback(ci):
            fv = fvs[ci % 2]
            start = ci * CH
            sizes = {}
            for qq in range(WPB):
                n_real = min(N - (qq * PW + start), CH)
                sizes.setdefault(max(n_real, 0), []).append(qq)
            dst = feat_hbm.at[pl.ds(obase + start, CH)]
            if list(sizes) == [CH]:
                return pltpu.async_copy(fv, dst, sem_o)
            for n_real, qs in sizes.items():
                if n_real == 0:
                    continue
                cond = functools.reduce(
                    lambda a, c: a | c, [q == qq for qq in qs])
                @pl.when(cond)
                def _():
                    pltpu.sync_copy(
                        fv.at[pl.ds(0, n_real)],
                        feat_hbm.at[pl.ds(obase + start, n_real)])
            return None

        hi = {0: load_idx(0)}
        if CHUNKS > 1:
            hi[1] = load_idx(1)
        hi[0].wait()
        hg = {0: fire_gather(0)}
        ho = {}
        for ci in range(CHUNKS):
            for cp in hg[ci]:
                cp.wait()
            if ci >= 1 and ho[ci - 1] is not None:
                ho[ci - 1].wait()
            if ci + 1 < CHUNKS:
                hi[ci + 1].wait()
                hg[ci + 1] = fire_gather(ci + 1)
                if ci + 2 < CHUNKS:
                    hi[ci + 2] = load_idx(ci + 2)
            ho[ci] = writeback(ci)
        if ho[CHUNKS - 1] is not None:
            ho[CHUNKS - 1].wait()

    return sc_kernel


def kernel(x, img):
    B, N, _ = x.shape
    _, C, H, W = img.shape

    WPB = NW // B
    per_worker = -(-N // WPB)
    per_worker = -(-per_worker // CH) * CH
    Npad = per_worker * WPB

    xp = jnp.pad(x, ((0, 0), (0, Npad - N), (0, 0)))
    rows = (B * Npad) // LANE
    x0 = xp[..., 0].reshape(rows, LANE)
    x1 = xp[..., 1].reshape(rows, LANE)
    x2 = xp[..., 2].reshape(rows, LANE)

    CW = C
    imgT = jnp.swapaxes(img.reshape(B, C, H * W), 1, 2).reshape(B * H * W, C)
    table = jnp.pad(imgT, ((0, 8), (0, 0)))

    idx, valid2d = _make_tc_project(B, Npad, C, H, W)(x0, x1, x2)
    feat_flat = _make_sc_gather(B, Npad, N, CW)(table, idx)

    feat = feat_flat.reshape(B, N, CW)[..., :C]
    valid = valid2d.reshape(B, Npad)[:, :N, None].astype(jnp.int64)
    return (feat, valid)


# speedup vs baseline: 3.8873x; 2.7242x over previous
import functools

import jax
import jax.numpy as jnp
from jax import lax
from jax.experimental import pallas as pl
from jax.experimental.pallas import tpu as pltpu
from jax.experimental.pallas import tpu_sc as plsc

NC = 2
NS = 16
NW = NC * NS
LANE = 128
CH = 2560
NR = CH // LANE
MAGIC = 12582912.0
NZ = 16384


def _make_tc_project(B, Npad, C, H, W):
    rows = (B * Npad) // LANE
    rows_per_batch = Npad // LANE
    zero_base = B * H * W
    hW = float(0.5 * W)
    hH = float(0.5 * H)

    def body(x0_ref, x1_ref, z_ref, idx_ref, valid_ref):
        zf = z_ref[...]
        a = x0_ref[...].astype(jnp.bfloat16).astype(jnp.float32)
        b = x1_ref[...].astype(jnp.bfloat16).astype(jnp.float32)
        z = zf.astype(jnp.bfloat16).astype(jnp.float32)
        uf = (a * hW + z * hW) / z
        vf = (b * hH + z * hH) / z
        ur = (uf + MAGIC) - MAGIC
        vr = (vf + MAGIC) - MAGIC
        cond = ((ur > 0.0) & (ur < W) & (vr > 0.0) & (vr < H) & (zf > 0.0))
        ui = ur.astype(jnp.int32)
        vi = vr.astype(jnp.int32)
        rid = lax.broadcasted_iota(jnp.int32, (rows, LANE), 0)
        tbase = (rid // rows_per_batch) * (H * W)
        ind = ui + vi * H + tbase
        lid = lax.broadcasted_iota(jnp.int32, (rows, LANE), 1)
        zidx = zero_base + ((rid * LANE + lid) & (NZ - 1))
        idx_ref[...] = jnp.where(cond, ind, zidx)
        valid_ref[...] = jnp.where(cond, 1, 0).astype(jnp.int32)

    return pl.pallas_call(
        body,
        out_shape=(
            jax.ShapeDtypeStruct((rows, LANE), jnp.int32),
            jax.ShapeDtypeStruct((rows, LANE), jnp.int32),
        ),
    )


def _make_sc_gather(B, Npad, N, CW):
    PW = (B * Npad) // NW
    CHUNKS = PW // CH
    WPB = NW // B
    mesh = plsc.VectorSubcoreMesh(core_axis_name="c", subcore_axis_name="s")

    @functools.partial(
        pl.kernel,
        mesh=mesh,
        out_type=jax.ShapeDtypeStruct((B * N, CW), jnp.float32),
        scratch_types=[
            pltpu.VMEM((NR, LANE), jnp.int32),
            pltpu.VMEM((NR, LANE), jnp.int32),
            pltpu.VMEM((CH, CW), jnp.float32),
            pltpu.VMEM((CH, CW), jnp.float32),
            pltpu.SemaphoreType.DMA,
            pltpu.SemaphoreType.DMA,
            pltpu.SemaphoreType.DMA,
        ],
        compiler_params=pltpu.CompilerParams(use_tc_tiling_on_sc=False),
    )
    def sc_kernel(tab_hbm, idx_hbm, feat_hbm, idxv0, idxv1, featv0, featv1,
                  sem_i, sem_g, sem_o):
        wid = lax.axis_index("s") * NC + lax.axis_index("c")
        rbase = (wid * PW) // LANE
        b = wid // WPB
        q = wid % WPB
        obase = b * N + q * PW
        ivs = (idxv0, idxv1)
        fvs = (featv0, featv1)

        def load_idx(ci):
            return pltpu.async_copy(
                idx_hbm.at[pl.ds(rbase + ci * NR, NR)], ivs[ci % 2], sem_i)

        def fire_gather(ci):
            iv, fv = ivs[ci % 2], fvs[ci % 2]
            return [
                pltpu.async_copy(tab_hbm.at[iv.at[r]],
                                 fv.at[pl.ds(r * LANE, LANE)], sem_g)
                for r in range(NR)
            ]

        def writeback(ci):
            fv = fvs[ci % 2]
            start = ci * CH
            sizes = {}
            for qq in range(WPB):
                n_real = min(N - (qq * PW + start), CH)
                sizes.setdefault(max(n_real, 0), []).append(qq)
            dst = feat_hbm.at[pl.ds(obase + start, CH)]
            if list(sizes) == [CH]:
                return pltpu.async_copy(fv, dst, sem_o)
            for n_real, qs in sizes.items():
                if n_real == 0:
                    continue
                cond = functools.reduce(
                    lambda a, c: a | c, [q == qq for qq in qs])
                @pl.when(cond)
                def _():
                    pltpu.sync_copy(
                        fv.at[pl.ds(0, n_real)],
                        feat_hbm.at[pl.ds(obase + start, n_real)])
            return None

        hi = {0: load_idx(0)}
        if CHUNKS > 1:
            hi[1] = load_idx(1)
        hi[0].wait()
        hg = {0: fire_gather(0)}
        ho = {}
        for ci in range(CHUNKS):
            for cp in hg[ci]:
                cp.wait()
            if ci >= 1 and ho[ci - 1] is not None:
                ho[ci - 1].wait()
            if ci + 1 < CHUNKS:
                hi[ci + 1].wait()
                hg[ci + 1] = fire_gather(ci + 1)
                if ci + 2 < CHUNKS:
                    hi[ci + 2] = load_idx(ci + 2)
            ho[ci] = writeback(ci)
        if ho[CHUNKS - 1] is not None:
            ho[CHUNKS - 1].wait()

    return sc_kernel


def kernel(x, img):
    B, N, _ = x.shape
    _, C, H, W = img.shape

    WPB = NW // B
    per_worker = -(-N // WPB)
    per_worker = -(-per_worker // CH) * CH
    Npad = per_worker * WPB

    xp = jnp.pad(x, ((0, 0), (0, Npad - N), (0, 0)))
    rows = (B * Npad) // LANE
    x0 = xp[..., 0].reshape(rows, LANE)
    x1 = xp[..., 1].reshape(rows, LANE)
    x2 = xp[..., 2].reshape(rows, LANE)

    CW = C
    imgT = jnp.swapaxes(img.reshape(B, C, H * W), 1, 2).reshape(B * H * W, C)
    table = jnp.pad(imgT, ((0, NZ), (0, 0)))

    idx, valid2d = _make_tc_project(B, Npad, C, H, W)(x0, x1, x2)
    feat_flat = _make_sc_gather(B, Npad, N, CW)(table, idx)

    feat = feat_flat.reshape(B, N, CW)[..., :C]
    valid = valid2d.reshape(B, Npad)[:, :N, None].astype(jnp.int64)
    return (feat, valid)
